# baseline (speedup 1.0000x reference)
import jax
import jax.numpy as jnp
import numpy as np
from jax.experimental import pallas as pl
from jax.experimental.pallas import tpu as pltpu
from jax.experimental.pallas import tpu_sc as plsc

_FREQ_WIDTH = 27
_TIME_WIDTH = 0.05

_F_STARTS = [[94, 9], [89, 5], [46, 3], [24, 63], [98, 71], [88, 51], [42, 51], [7, 82], [29, 31], [65, 31], [89, 29], [28, 85], [57, 61], [55, 62], [66, 75], [72, 43], [12, 53], [43, 58], [88, 39], [57, 19], [92, 50], [20, 90], [80, 47], [0, 20], [61, 84], [53, 61], [87, 69], [101, 90], [39, 31], [58, 9], [6, 31], [12, 36], [96, 12], [75, 21], [23, 14], [52, 18], [35, 62], [10, 63], [52, 24], [19, 72], [94, 40], [76, 17], [85, 53], [82, 3], [81, 13], [8, 63], [59, 74], [23, 25], [96, 27], [17, 3], [55, 23], [85, 82], [83, 16], [45, 63], [4, 33], [66, 35], [62, 90], [19, 32], [26, 49], [14, 80], [19, 66], [76, 68], [101, 65], [31, 69]]

_T_STARTS = [[1934, 446, 1804, 584, 1654], [1242, 982, 1093, 1865, 487], [1151, 1260, 789, 1656, 1254], [18, 501, 1636, 187, 1345], [827, 1275, 1795, 185, 690], [920, 196, 932, 1937, 1353], [864, 694, 1914, 846, 1885], [1627, 1306, 1698, 395, 605], [106, 679, 1671, 460, 334], [409, 1443, 1452, 1865, 482], [956, 1034, 309, 1497, 1375], [167, 600, 930, 34, 680], [1665, 1595, 1521, 459, 378], [191, 1943, 355, 480, 919], [39, 1229, 218, 1723, 1902], [1655, 108, 717, 120, 627], [1004, 462, 1569, 1301, 1374], [1178, 1592, 1072, 456, 104], [779, 889, 1258, 287, 299], [328, 400, 1614, 1758, 1085], [1789, 340, 1427, 1248, 1428], [176, 185, 21, 1497, 1357], [228, 1019, 675, 1196, 865], [310, 908, 1161, 800, 30], [583, 1608, 1574, 291, 275], [1541, 1631, 1804, 174, 850], [488, 659, 1860, 470, 977], [1063, 1200, 50, 342, 1116], [716, 1417, 1229, 1877, 268], [1632, 1905, 1849, 975, 447], [523, 723, 1610, 566, 909], [695, 20, 657, 497, 1211], [1022, 223, 73, 83, 978], [1627, 1498, 241, 1403, 768], [1336, 1740, 1010, 527, 1270], [1077, 1898, 143, 1503, 1933], [185, 774, 29, 57, 1483], [935, 1469, 1757, 474, 17], [981, 806, 524, 170, 307], [1080, 125, 1747, 106, 746], [1729, 252, 555, 644, 810], [761, 1286, 1564, 1031, 1126], [464, 895, 1847, 1732, 1765], [259, 464, 466, 1038, 1177], [1871, 905, 202, 90, 307], [745, 151, 871, 1084, 554], [191, 1079, 1921, 103, 1577], [873, 1729, 624, 1873, 1764], [68, 1628, 867, 447, 737], [1810, 627, 1892, 641, 236], [1379, 1305, 481, 0, 1765], [1498, 1494, 289, 629, 1769], [1486, 488, 1101, 1637, 3], [1486, 691, 975, 1094, 253], [671, 1584, 1859, 1462, 303], [944, 704, 429, 1118, 1225], [1271, 1303, 1248, 1136, 18], [1558, 786, 1536, 1737, 1357], [247, 610, 156, 1025, 1116], [311, 1695, 1041, 1559, 1651], [1702, 871, 297, 534, 954], [1487, 1346, 1136, 334, 1804], [1096, 1663, 853, 196, 224], [1643, 903, 1234, 1795, 386]]

_SC_SAMPLES = 64
_TC_SAMPLES_PER_BLOCK = 8


def _keep_tables(B, F, T, tw):
    fk = np.ones((B, F), np.float32)
    tk = np.ones((B, T), np.float32)
    for b in range(B):
        for s in _F_STARTS[b]:
            fk[b, s:s + _FREQ_WIDTH] = 0.0
        for s in _T_STARTS[b]:
            tk[b, s:s + tw] = 0.0
    fkb = np.repeat(fk.reshape(B * F, 1), 128, axis=1)
    return fkb, tk


def kernel(input_spec):
    B, F, T = input_spec.shape
    tw = int(_TIME_WIDTH * T)
    fkb_np, tk_np = _keep_tables(B, F, T, tw)
    fkb = jnp.asarray(fkb_np)
    tk = jnp.asarray(tk_np)
    x2 = input_spec.reshape(B * F, T)

    S = B - _SC_SAMPLES
    spb = _TC_SAMPLES_PER_BLOCK

    tc_mesh = pltpu.create_tensorcore_mesh("tc_core")
    sc_mesh = plsc.VectorSubcoreMesh(
        core_axis_name="core", subcore_axis_name="subcore")

    def tc_body(x_hbm, fkb_hbm, tk_hbm, o_hbm):
        def inner(x_v, fkb_v, tk_v, o_v):
            for s in range(spb):
                rows = slice(s * F, (s + 1) * F)
                fk_col = fkb_v[rows, 0:1]
                tk_row = tk_v[s:s + 1, :]
                keep = (fk_col * tk_row) > 0.0
                o_v[rows, :] = jnp.where(
                    keep, x_v[rows, :], jnp.float32(0.0))

        pltpu.emit_pipeline(
            inner,
            grid=(S // spb,),
            in_specs=[
                pl.BlockSpec((spb * F, T), lambda j: (j, 0)),
                pl.BlockSpec((spb * F, 128), lambda j: (j, 0)),
                pl.BlockSpec((spb, T), lambda j: (j, 0)),
            ],
            out_specs=[pl.BlockSpec((spb * F, T), lambda j: (j, 0))],
        )(x_hbm, fkb_hbm, tk_hbm, o_hbm)

    def sc_body(x_hbm, fkb_hbm, tk_hbm, o_hbm):
        base = S * (F // 8)

        def inner(x_v, fkb_v, tk_v, o_v):
            @pl.loop(0, 8)
            def _(r):
                @pl.loop(0, T, step=16)
                def _(c):
                    xs = x_v.at[r, pl.ds(c, 16)][...]
                    tks = tk_v.at[0, pl.ds(c, 16)][...]
                    fks = fkb_v.at[r, pl.ds(0, 16)][...]
                    keep = (tks * fks) > 0.0
                    o_v.at[r, pl.ds(c, 16)][...] = jnp.where(
                        keep, xs, jnp.float32(0.0))

        pltpu.emit_pipeline(
            inner,
            grid=(_SC_SAMPLES * (F // 8),),
            in_specs=[
                pl.BlockSpec((8, T), lambda j: (base + j, 0)),
                pl.BlockSpec((8, 128), lambda j: (base + j, 0)),
                pl.BlockSpec((1, T), lambda j: (S + j // (F // 8), 0)),
            ],
            out_specs=[pl.BlockSpec((8, T), lambda j: (base + j, 0))],
            core_axis_name=("core", "subcore"),
            dimension_semantics=(pltpu.PARALLEL,),
        )(x_hbm, fkb_hbm, tk_hbm, o_hbm)

    hbm = pltpu.MemorySpace.HBM
    out = pl.kernel(
        sc_body,
        out_type=pltpu.HBM((B * F, T), input_spec.dtype),
        mesh=sc_mesh,
    )(
        pltpu.with_memory_space_constraint(x2, hbm),
        pltpu.with_memory_space_constraint(fkb, hbm),
        pltpu.with_memory_space_constraint(tk, hbm),
    )
    return out.reshape(B, F, T)
